# baseline (speedup 1.0000x reference)
import functools

import jax
import jax.numpy as jnp
from jax import lax
from jax.experimental import pallas as pl
from jax.experimental.pallas import tpu as pltpu
from jax.experimental.pallas import tpu_sc as plsc

N_WORDS = 4096
N_SAMPLES = 256
PATH_LEN = 128
NC = 2
NS = 16
NW = NC * NS
SPW = N_SAMPLES // NW
DPW = N_WORDS // NW


def _body(bigram_hbm, bias_hbm, start_hbm, end_hbm, samples_hbm, out_hbm,
          samp_v, idx_v, gb_v, gc_v, didx_v, dgb_v, dbb_v,
          sidx_v, eidx_v, sg_v, eg_v, part_v, sem1, sem2, sem3):
    wid = lax.axis_index("s") * NC + lax.axis_index("c")
    base = wid * SPW
    iota = lax.iota(jnp.int32, 16)

    def phys(r, c):
        return ((jnp.right_shift(r, 3) << 15) + (jnp.bitwise_and(r, 7) << 7)
                + (jnp.right_shift(c, 7) << 10) + jnp.bitwise_and(c, 127))

    h_samp = pltpu.async_copy(samples_hbm.at[pl.ds(base, SPW)], samp_v, sem2)
    for k in range(8):
        ivec = iota + (wid * DPW + k * 16)
        d = phys(ivec, jnp.minimum(ivec + 1, N_WORDS - 1))
        d = jnp.where(ivec < N_WORDS - 1, d, wid)
        didx_v[pl.ds(k * 16, 16)] = d
    h_dgb = pltpu.async_copy(bigram_hbm.at[didx_v], dgb_v, sem1)
    h_dbb = pltpu.async_copy(bias_hbm.at[didx_v], dbb_v, sem1)

    h_samp.wait()

    for s in range(SPW):
        srow = jnp.full((16,), s, jnp.int32)
        for k in range(8):
            h = samp_v[s, pl.ds(k * 16, 16)]
            tcol = jnp.minimum(iota + (k * 16 + 1), PATH_LEN - 1)
            t = plsc.load_gather(samp_v, [srow, tcol])
            idx_v[s, pl.ds(k * 16, 16)] = phys(h, t)
    hb = [pltpu.async_copy(bigram_hbm.at[idx_v.at[s]], gb_v.at[s], sem2)
          for s in range(SPW)]
    hc = [pltpu.async_copy(bias_hbm.at[idx_v.at[s]], gc_v.at[s], sem3)
          for s in range(SPW)]

    rows8 = lax.bitwise_and(iota, jnp.full((16,), SPW - 1, jnp.int32))
    z16 = jnp.zeros((16,), jnp.int32)
    sidx = plsc.load_gather(samp_v, [rows8, z16])
    eidx = plsc.load_gather(samp_v, [rows8, z16 + (PATH_LEN - 1)])
    sidx_v[...] = jnp.where(iota < SPW, sidx, wid)
    eidx_v[...] = jnp.where(iota < SPW, eidx, N_WORDS - 1 - wid)
    hb.append(pltpu.async_copy(start_hbm.at[sidx_v], sg_v, sem2))
    hb.append(pltpu.async_copy(end_hbm.at[eidx_v], eg_v, sem2))

    f16 = lambda c: jnp.where(c, 1.0, 0.0).astype(jnp.float32)
    m15 = f16(iota < 15)
    m8 = f16(iota < SPW)

    h_dgb.wait()
    h_dbb.wait()
    dacc = jnp.zeros((16,), jnp.float32)
    for k in range(8):
        ivec = iota + (wid * DPW + k * 16)
        mk = f16(ivec < N_WORDS - 1)
        dacc = dacc + (dgb_v[pl.ds(k * 16, 16)] + dbb_v[pl.ds(k * 16, 16)]) * mk
    d = jnp.sum(dacc)

    for h in hb:
        h.wait()
    srB = jnp.zeros((16,), jnp.float32)
    for s in range(SPW):
        accb = gb_v[s, pl.ds(7 * 16, 16)] * m15
        for k in range(7):
            accb = accb + gb_v[s, pl.ds(k * 16, 16)]
        srB = jnp.where(iota == s, jnp.sum(accb), srB)
    sr = srB + sg_v[...] * m8 + eg_v[...] * m8
    t1 = jnp.sum(sr)
    t2 = jnp.sum(sr * sr)
    s0eL = jnp.sum((sg_v[...] + eg_v[...]) * f16(iota == 8))
    d = d + s0eL * jnp.where(wid == 0, 1.0, 0.0)

    for h in hc:
        h.wait()
    pb = jnp.zeros((16,), jnp.float32)
    for s in range(SPW):
        accc = gc_v[s, pl.ds(7 * 16, 16)] * m15
        for k in range(7):
            accc = accc + gc_v[s, pl.ds(k * 16, 16)]
        pb = jnp.where(iota == s, jnp.sum(accc), pb)
    t3 = jnp.sum(sr * pb)

    part_v[...] = (f16(iota == 0) * t1 + f16(iota == 1) * t2
                   + f16(iota == 2) * t3 + f16(iota == 3) * d)
    pltpu.sync_copy(part_v, out_hbm.at[wid])


@jax.jit
def kernel(bigram, start, end, bigram_bias, samples):
    mesh = plsc.VectorSubcoreMesh(core_axis_name="c", subcore_axis_name="s")
    launch = functools.partial(
        pl.kernel,
        mesh=mesh,
        compiler_params=pltpu.CompilerParams(needs_layout_passes=False),
        out_type=jax.ShapeDtypeStruct((NW, 16), jnp.float32),
        scratch_types=[
            pltpu.VMEM((SPW, PATH_LEN), jnp.int32),
            pltpu.VMEM((SPW, PATH_LEN), jnp.int32),
            pltpu.VMEM((SPW, PATH_LEN), jnp.float32),
            pltpu.VMEM((SPW, PATH_LEN), jnp.float32),
            pltpu.VMEM((DPW,), jnp.int32),
            pltpu.VMEM((DPW,), jnp.float32),
            pltpu.VMEM((DPW,), jnp.float32),
            pltpu.VMEM((16,), jnp.int32),
            pltpu.VMEM((16,), jnp.int32),
            pltpu.VMEM((16,), jnp.float32),
            pltpu.VMEM((16,), jnp.float32),
            pltpu.VMEM((16,), jnp.float32),
            pltpu.SemaphoreType.DMA,
            pltpu.SemaphoreType.DMA,
            pltpu.SemaphoreType.DMA,
        ],
    )(_body)
    def tiled_bytes_view(x):
        return x.reshape(512, 8, 32, 128).transpose(0, 2, 1, 3).reshape(-1)

    parts = launch(tiled_bytes_view(bigram), tiled_bytes_view(bigram_bias),
                   start, end, samples)
    p = parts.sum(axis=0)
    loss = (p[1] + p[2]) / p[0] - p[3]
    return (loss, 0)


# speedup vs baseline: 1.0378x; 1.0378x over previous
import functools

import jax
import jax.numpy as jnp
from jax import lax
from jax.experimental import pallas as pl
from jax.experimental.pallas import tpu as pltpu
from jax.experimental.pallas import tpu_sc as plsc

N_WORDS = 4096
N_SAMPLES = 256
PATH_LEN = 128
NC = 2
NS = 16
NW = NC * NS
SPW = N_SAMPLES // NW
DPW = N_WORDS // NW


def _body(bigram_hbm, bias_hbm, start_hbm, end_hbm, samples_hbm, out_hbm,
          samp_v, idx_v, gb_v, gc_v, didx_v, dgb_v, dbb_v,
          sidx_v, eidx_v, sg_v, eg_v, part_v, sem1, sem2, sem3):
    wid = lax.axis_index("s") * NC + lax.axis_index("c")
    base = wid * SPW
    iota = lax.iota(jnp.int32, 16)

    def phys(r, c):
        return ((jnp.right_shift(r, 3) << 15) + (jnp.bitwise_and(r, 7) << 7)
                + (jnp.right_shift(c, 7) << 10) + jnp.bitwise_and(c, 127))

    h_samp = [
        pltpu.async_copy(samples_hbm.at[base + s], samp_v.at[s, pl.ds(0, PATH_LEN)], sem2)
        for s in range(SPW)
    ]
    for k in range(8):
        ivec = iota + (wid * DPW + k * 16)
        d = phys(ivec, jnp.minimum(ivec + 1, N_WORDS - 1))
        d = jnp.where(ivec < N_WORDS - 1, d, wid)
        didx_v[pl.ds(k * 16, 16)] = d
    h_dgb = pltpu.async_copy(bigram_hbm.at[didx_v], dgb_v, sem1)
    h_dbb = pltpu.async_copy(bias_hbm.at[didx_v], dbb_v, sem1)

    for h in h_samp:
        h.wait()

    z16 = jnp.zeros((16,), jnp.int32)
    for s in range(SPW):
        samp_v[s, pl.ds(PATH_LEN, 16)] = z16
        for k in range(8):
            h = samp_v[s, pl.ds(k * 16, 16)]
            t = samp_v[s, pl.ds(k * 16 + 1, 16)]
            idx_v[pl.ds(s * PATH_LEN + k * 16, 16)] = phys(h, t)
    hb = [pltpu.async_copy(bigram_hbm.at[idx_v], gb_v, sem2)]
    hc = [pltpu.async_copy(bias_hbm.at[idx_v], gc_v, sem3)]

    sidx = jnp.full((16,), wid, jnp.int32)
    eidx = jnp.full((16,), N_WORDS - 1 - wid, jnp.int32)
    for s in range(SPW):
        sidx = jnp.where(iota == s, samp_v[s, pl.ds(0, 16)][0], sidx)
        eidx = jnp.where(iota == s, samp_v[s, pl.ds(PATH_LEN - 16, 16)][15], eidx)
    sidx_v[...] = sidx
    eidx_v[...] = eidx
    hb.append(pltpu.async_copy(start_hbm.at[sidx_v], sg_v, sem2))
    hb.append(pltpu.async_copy(end_hbm.at[eidx_v], eg_v, sem2))

    f16 = lambda c: jnp.where(c, 1.0, 0.0).astype(jnp.float32)
    m15 = f16(iota < 15)
    m8 = f16(iota < SPW)

    h_dgb.wait()
    h_dbb.wait()
    dacc = jnp.zeros((16,), jnp.float32)
    for k in range(8):
        ivec = iota + (wid * DPW + k * 16)
        mk = f16(ivec < N_WORDS - 1)
        dacc = dacc + (dgb_v[pl.ds(k * 16, 16)] + dbb_v[pl.ds(k * 16, 16)]) * mk
    d = jnp.sum(dacc)

    for h in hb:
        h.wait()
    srB = jnp.zeros((16,), jnp.float32)
    for s in range(SPW):
        accb = gb_v[pl.ds(s * PATH_LEN + 7 * 16, 16)] * m15
        for k in range(7):
            accb = accb + gb_v[pl.ds(s * PATH_LEN + k * 16, 16)]
        srB = jnp.where(iota == s, jnp.sum(accb), srB)
    sr = srB + sg_v[...] * m8 + eg_v[...] * m8
    t1 = jnp.sum(sr)
    t2 = jnp.sum(sr * sr)
    s0eL = jnp.sum((sg_v[...] + eg_v[...]) * f16(iota == 8))
    d = d + s0eL * jnp.where(wid == 0, 1.0, 0.0)

    for h in hc:
        h.wait()
    pb = jnp.zeros((16,), jnp.float32)
    for s in range(SPW):
        accc = gc_v[pl.ds(s * PATH_LEN + 7 * 16, 16)] * m15
        for k in range(7):
            accc = accc + gc_v[pl.ds(s * PATH_LEN + k * 16, 16)]
        pb = jnp.where(iota == s, jnp.sum(accc), pb)
    t3 = jnp.sum(sr * pb)

    part_v[...] = (f16(iota == 0) * t1 + f16(iota == 1) * t2
                   + f16(iota == 2) * t3 + f16(iota == 3) * d)
    pltpu.sync_copy(part_v, out_hbm.at[wid])


@jax.jit
def kernel(bigram, start, end, bigram_bias, samples):
    mesh = plsc.VectorSubcoreMesh(core_axis_name="c", subcore_axis_name="s")
    launch = functools.partial(
        pl.kernel,
        mesh=mesh,
        compiler_params=pltpu.CompilerParams(needs_layout_passes=False),
        out_type=jax.ShapeDtypeStruct((NW, 16), jnp.float32),
        scratch_types=[
            pltpu.VMEM((SPW, PATH_LEN + 16), jnp.int32),
            pltpu.VMEM((SPW * PATH_LEN,), jnp.int32),
            pltpu.VMEM((SPW * PATH_LEN,), jnp.float32),
            pltpu.VMEM((SPW * PATH_LEN,), jnp.float32),
            pltpu.VMEM((DPW,), jnp.int32),
            pltpu.VMEM((DPW,), jnp.float32),
            pltpu.VMEM((DPW,), jnp.float32),
            pltpu.VMEM((16,), jnp.int32),
            pltpu.VMEM((16,), jnp.int32),
            pltpu.VMEM((16,), jnp.float32),
            pltpu.VMEM((16,), jnp.float32),
            pltpu.VMEM((16,), jnp.float32),
            pltpu.SemaphoreType.DMA,
            pltpu.SemaphoreType.DMA,
            pltpu.SemaphoreType.DMA,
        ],
    )(_body)
    def tiled_bytes_view(x):
        return x.reshape(512, 8, 32, 128).transpose(0, 2, 1, 3).reshape(-1)

    parts = launch(tiled_bytes_view(bigram), tiled_bytes_view(bigram_bias),
                   start, end, samples)
    p = parts.sum(axis=0)
    loss = (p[1] + p[2]) / p[0] - p[3]
    return (loss, 0)
